# baseline (speedup 1.0000x reference)
import functools

import jax
import jax.numpy as jnp
from jax.experimental import pallas as pl
from jax.experimental.pallas import tpu as pltpu

TOP_K = 2
CAPACITY_FACTOR = 1.25
ENTROPY_THRESHOLD = 1.0
EPS = 1e-8

LB = 256
DC = 512
LB3 = 512


def _phase1_kernel(params_ref, x_ref, gw_ref, gb_ref,
                   disp_ref, pooled_ref, counts_ref, gates_ref, ents_ref,
                   run_ref, *, capacity, num_lb):
    lb = pl.program_id(1)

    @pl.when(lb == 0)
    def _init():
        run_ref[...] = jnp.zeros_like(run_ref)
        pooled_ref[...] = jnp.zeros_like(pooled_ref)
        counts_ref[...] = jnp.zeros_like(counts_ref)
        gates_ref[...] = jnp.zeros_like(gates_ref)
        ents_ref[...] = jnp.zeros_like(ents_ref)

    xb = x_ref[0]
    gw = gw_ref[...]
    E = gw.shape[0]
    t = params_ref[0]
    ew = params_ref[1]
    cw = params_ref[2]
    uw = params_ref[3]

    logits = jax.lax.dot_general(xb, gw, (((1,), (1,)), ((), ())),
                                 preferred_element_type=jnp.float32)
    logits = (logits + gb_ref[...]) / t

    m = jnp.max(logits, axis=-1, keepdims=True)
    ex = jnp.exp(logits - m)
    p = ex / jnp.sum(ex, axis=-1, keepdims=True)

    ent = -jnp.sum(p * jnp.log(p + EPS), axis=-1, keepdims=True)
    mean = jnp.mean(p, axis=-1, keepdims=True)
    var = jnp.sum((p - mean) ** 2, axis=-1, keepdims=True) / (E - 1)
    conf = 1.0 / (var + EPS)
    ent_norm = jax.nn.sigmoid(ent / ENTROPY_THRESHOLD)
    af = jax.nn.sigmoid(ew * ent_norm + cw * conf + uw * var)

    mp = p * (1.0 + af)
    mp = mp / jnp.sum(mp, axis=-1, keepdims=True)

    e_iota = jax.lax.broadcasted_iota(jnp.int32, mp.shape, 1)
    i1 = jnp.argmax(mp, axis=-1)
    v1 = jnp.max(mp, axis=-1)
    mask1 = (e_iota == i1[:, None])
    mp2 = jnp.where(mask1, -jnp.inf, mp)
    i2 = jnp.argmax(mp2, axis=-1)
    v2 = jnp.max(mp2, axis=-1)
    mask2 = (e_iota == i2[:, None])
    wn = jnp.clip(v1 + v2, 1e-9, None)
    disp = (v1 / wn)[:, None] * mask1.astype(jnp.float32) \
         + (v2 / wn)[:, None] * mask2.astype(jnp.float32)

    assign = (disp > 0).astype(jnp.float32)
    r = jax.lax.broadcasted_iota(jnp.int32, (assign.shape[0], assign.shape[0]), 0)
    c = jax.lax.broadcasted_iota(jnp.int32, (assign.shape[0], assign.shape[0]), 1)
    tri = (r >= c).astype(jnp.float32)
    csum = jax.lax.dot_general(tri, assign, (((1,), (0,)), ((), ())),
                               preferred_element_type=jnp.float32)
    positions = run_ref[...] + csum - 1.0
    keep = (positions < float(capacity)).astype(jnp.float32)
    disp = disp * keep
    run_ref[...] += jnp.sum(assign, axis=0, keepdims=True)

    disp_ref[0] = disp
    pooled_ref[0] += jax.lax.dot_general(disp, xb, (((0,), (0,)), ((), ())),
                                         preferred_element_type=jnp.float32)
    counts_ref[0] += jnp.sum(disp, axis=0, keepdims=True)
    gates_ref[0] += jnp.sum(p, axis=0, keepdims=True)
    ents_ref[0] += jnp.broadcast_to(jnp.sum(ent, keepdims=True), ents_ref[0].shape)


def _phase2_kernel(pooled_ref, w_ref, b_ref, invc_ref, out_ref):
    inp = pooled_ref[0]
    w = w_ref[0]
    acc = jax.lax.dot_general(inp, w, (((1,), (1,)), ((), ())),
                              preferred_element_type=jnp.float32)
    out_ref[0] = acc * invc_ref[0] + b_ref[0]


def _phase3_kernel(disp_ref, eo_ref, out_ref):
    out_ref[0] = jax.lax.dot_general(disp_ref[0], eo_ref[0],
                                     (((1,), (0,)), ((), ())),
                                     preferred_element_type=jnp.float32)


def kernel(x, gate_W, gate_b, expert_W, expert_b, temperature,
           entropy_weight, confidence_weight, uncertainty_weight):
    B, L, D = x.shape
    E = gate_W.shape[0]
    capacity = int(CAPACITY_FACTOR * (B * L / max(1, E)) + 0.9999)
    num_lb = L // LB

    params = jnp.concatenate([temperature, entropy_weight,
                              confidence_weight, uncertainty_weight])
    gb2 = gate_b.reshape(1, E)

    disp, pooled, counts, gates, ents = pl.pallas_call(
        functools.partial(_phase1_kernel, capacity=capacity, num_lb=num_lb),
        grid=(B, num_lb),
        in_specs=[
            pl.BlockSpec(memory_space=pltpu.SMEM),
            pl.BlockSpec((1, LB, D), lambda b, l: (b, l, 0)),
            pl.BlockSpec((E, D), lambda b, l: (0, 0)),
            pl.BlockSpec((1, E), lambda b, l: (0, 0)),
        ],
        out_specs=[
            pl.BlockSpec((1, LB, E), lambda b, l: (b, l, 0)),
            pl.BlockSpec((1, E, D), lambda b, l: (b, 0, 0)),
            pl.BlockSpec((1, 1, E), lambda b, l: (b, 0, 0)),
            pl.BlockSpec((1, 1, E), lambda b, l: (b, 0, 0)),
            pl.BlockSpec((1, 1, E), lambda b, l: (b, 0, 0)),
        ],
        out_shape=[
            jax.ShapeDtypeStruct((B, L, E), jnp.float32),
            jax.ShapeDtypeStruct((B, E, D), jnp.float32),
            jax.ShapeDtypeStruct((B, 1, E), jnp.float32),
            jax.ShapeDtypeStruct((B, 1, E), jnp.float32),
            jax.ShapeDtypeStruct((B, 1, E), jnp.float32),
        ],
        scratch_shapes=[pltpu.VMEM((1, E), jnp.float32)],
    )(params, x, gate_W, gb2)

    counts2 = counts[:, 0, :]
    invc = 1.0 / jnp.clip(counts2, 1.0, None)
    invc_t = invc.T[..., None]
    pooled_t = jnp.swapaxes(pooled, 0, 1)
    eb3 = expert_b[:, None, :]

    num_dc = D // DC
    eo = pl.pallas_call(
        _phase2_kernel,
        grid=(E, num_dc),
        in_specs=[
            pl.BlockSpec((1, B, D), lambda e, d: (e, 0, 0)),
            pl.BlockSpec((1, DC, D), lambda e, d: (e, d, 0)),
            pl.BlockSpec((1, 1, DC), lambda e, d: (e, 0, d)),
            pl.BlockSpec((1, B, 1), lambda e, d: (e, 0, 0)),
        ],
        out_specs=pl.BlockSpec((1, B, DC), lambda e, d: (e, 0, d)),
        out_shape=jax.ShapeDtypeStruct((E, B, D), jnp.float32),
    )(pooled_t, expert_W, eb3, invc_t)

    eo_t = jnp.swapaxes(eo, 0, 1)

    num_lb3 = L // LB3
    out = pl.pallas_call(
        _phase3_kernel,
        grid=(B, num_lb3),
        in_specs=[
            pl.BlockSpec((1, LB3, E), lambda b, l: (b, l, 0)),
            pl.BlockSpec((1, E, D), lambda b, l: (b, 0, 0)),
        ],
        out_specs=pl.BlockSpec((1, LB3, D), lambda b, l: (b, l, 0)),
        out_shape=jax.ShapeDtypeStruct((B, L, D), jnp.float32),
    )(disp, eo_t)

    util = jnp.sum(counts2, axis=0) / (B * L)
    diversity_loss = -jnp.var(util, ddof=1) * 0.01
    mean_gate = gates[:, 0, :] / L
    aux_loss = jnp.var(mean_gate) * E + diversity_loss
    avg_ent = jnp.sum(ents[:, 0, 0]) / (B * L)
    aux_loss = aux_loss + (avg_ent - ENTROPY_THRESHOLD) ** 2 * 0.01
    return (out, aux_loss)


# speedup vs baseline: 1.2971x; 1.2971x over previous
import functools

import jax
import jax.numpy as jnp
from jax.experimental import pallas as pl
from jax.experimental.pallas import tpu as pltpu

TOP_K = 2
CAPACITY_FACTOR = 1.25
ENTROPY_THRESHOLD = 1.0
EPS = 1e-8

LB = 512
DC = 512
LB3 = 512


def _phase1_kernel(params_ref, x_ref, gw_ref, gb_ref,
                   disp_ref, pooled_ref, counts_ref, gates_ref, ents_ref,
                   run_ref, *, capacity):
    lb = pl.program_id(1)

    @pl.when(lb == 0)
    def _init():
        run_ref[...] = jnp.zeros_like(run_ref)
        pooled_ref[...] = jnp.zeros_like(pooled_ref)
        counts_ref[...] = jnp.zeros_like(counts_ref)
        gates_ref[...] = jnp.zeros_like(gates_ref)
        ents_ref[...] = jnp.zeros_like(ents_ref)

    xb = x_ref[0]
    gw = gw_ref[...]
    E = gw.shape[0]
    t = params_ref[0]
    ew = params_ref[1]
    cw = params_ref[2]
    uw = params_ref[3]

    logits = jax.lax.dot_general(gw, xb, (((1,), (1,)), ((), ())),
                                 preferred_element_type=jnp.float32)
    logits = (logits + gb_ref[...]) / t

    m = jnp.max(logits, axis=0, keepdims=True)
    ex = jnp.exp(logits - m)
    p = ex / jnp.sum(ex, axis=0, keepdims=True)

    ent = -jnp.sum(p * jnp.log(p + EPS), axis=0, keepdims=True)
    mean = jnp.mean(p, axis=0, keepdims=True)
    var = jnp.sum((p - mean) ** 2, axis=0, keepdims=True) / (E - 1)
    conf = 1.0 / (var + EPS)
    ent_norm = jax.nn.sigmoid(ent / ENTROPY_THRESHOLD)
    af = jax.nn.sigmoid(ew * ent_norm + cw * conf + uw * var)

    mp = p * (1.0 + af)
    mp = mp / jnp.sum(mp, axis=0, keepdims=True)

    e_iota = jax.lax.broadcasted_iota(jnp.int32, mp.shape, 0)
    m1 = jnp.max(mp, axis=0, keepdims=True)
    i1 = jnp.min(jnp.where(mp == m1, e_iota, E), axis=0, keepdims=True)
    mask1 = (e_iota == i1)
    mp2 = jnp.where(mask1, -jnp.inf, mp)
    m2 = jnp.max(mp2, axis=0, keepdims=True)
    i2 = jnp.min(jnp.where(mp2 == m2, e_iota, E), axis=0, keepdims=True)
    mask2 = (e_iota == i2)
    wn = jnp.clip(m1 + m2, 1e-9, None)
    disp = mask1.astype(jnp.float32) * (m1 / wn) \
         + mask2.astype(jnp.float32) * (m2 / wn)

    assign = (disp > 0).astype(jnp.float32)
    n = assign.shape[1]
    r = jax.lax.broadcasted_iota(jnp.int32, (n, n), 0)
    c = jax.lax.broadcasted_iota(jnp.int32, (n, n), 1)
    triu = (r <= c).astype(jnp.float32)
    csum = jax.lax.dot_general(assign, triu, (((1,), (0,)), ((), ())),
                               preferred_element_type=jnp.float32)
    positions = run_ref[...] + csum - 1.0
    keep = (positions < float(capacity)).astype(jnp.float32)
    disp = disp * keep
    run_ref[...] += jnp.sum(assign, axis=1, keepdims=True)

    disp_ref[0] = disp
    pooled_ref[0] += jax.lax.dot_general(disp, xb, (((1,), (0,)), ((), ())),
                                         preferred_element_type=jnp.float32)
    counts_ref[0] += jnp.sum(disp, axis=1, keepdims=True)
    gates_ref[0] += jnp.sum(p, axis=1, keepdims=True)
    ents_ref[0] += jnp.broadcast_to(jnp.sum(ent, keepdims=True), ents_ref[0].shape)


def _phase2_kernel(pooled_ref, w_ref, b_ref, invc_ref, out_ref):
    inp = pooled_ref[0]
    w = w_ref[0]
    acc = jax.lax.dot_general(inp, w, (((1,), (1,)), ((), ())),
                              preferred_element_type=jnp.float32)
    out_ref[0] = acc * invc_ref[0] + b_ref[0]


def _phase3_kernel(disp_ref, eo_ref, out_ref):
    out_ref[0] = jax.lax.dot_general(disp_ref[0], eo_ref[0],
                                     (((0,), (0,)), ((), ())),
                                     preferred_element_type=jnp.float32)


def kernel(x, gate_W, gate_b, expert_W, expert_b, temperature,
           entropy_weight, confidence_weight, uncertainty_weight):
    B, L, D = x.shape
    E = gate_W.shape[0]
    capacity = int(CAPACITY_FACTOR * (B * L / max(1, E)) + 0.9999)
    num_lb = L // LB

    params = jnp.concatenate([temperature, entropy_weight,
                              confidence_weight, uncertainty_weight])
    gb2 = gate_b.reshape(E, 1)

    disp, pooled, counts, gates, ents = pl.pallas_call(
        functools.partial(_phase1_kernel, capacity=capacity),
        grid=(B, num_lb),
        in_specs=[
            pl.BlockSpec(memory_space=pltpu.SMEM),
            pl.BlockSpec((1, LB, D), lambda b, l: (b, l, 0)),
            pl.BlockSpec((E, D), lambda b, l: (0, 0)),
            pl.BlockSpec((E, 1), lambda b, l: (0, 0)),
        ],
        out_specs=[
            pl.BlockSpec((1, E, LB), lambda b, l: (b, 0, l)),
            pl.BlockSpec((1, E, D), lambda b, l: (b, 0, 0)),
            pl.BlockSpec((1, E, 1), lambda b, l: (b, 0, 0)),
            pl.BlockSpec((1, E, 1), lambda b, l: (b, 0, 0)),
            pl.BlockSpec((1, E, 1), lambda b, l: (b, 0, 0)),
        ],
        out_shape=[
            jax.ShapeDtypeStruct((B, E, L), jnp.float32),
            jax.ShapeDtypeStruct((B, E, D), jnp.float32),
            jax.ShapeDtypeStruct((B, E, 1), jnp.float32),
            jax.ShapeDtypeStruct((B, E, 1), jnp.float32),
            jax.ShapeDtypeStruct((B, E, 1), jnp.float32),
        ],
        scratch_shapes=[pltpu.VMEM((E, 1), jnp.float32)],
    )(params, x, gate_W, gb2)

    counts2 = counts[:, :, 0]
    invc = 1.0 / jnp.clip(counts2, 1.0, None)
    invc_t = invc.T[..., None]
    pooled_t = jnp.swapaxes(pooled, 0, 1)
    eb3 = expert_b[:, None, :]

    num_dc = D // DC
    eo = pl.pallas_call(
        _phase2_kernel,
        grid=(E, num_dc),
        in_specs=[
            pl.BlockSpec((1, B, D), lambda e, d: (e, 0, 0)),
            pl.BlockSpec((1, DC, D), lambda e, d: (e, d, 0)),
            pl.BlockSpec((1, 1, DC), lambda e, d: (e, 0, d)),
            pl.BlockSpec((1, B, 1), lambda e, d: (e, 0, 0)),
        ],
        out_specs=pl.BlockSpec((1, B, DC), lambda e, d: (e, 0, d)),
        out_shape=jax.ShapeDtypeStruct((E, B, D), jnp.float32),
    )(pooled_t, expert_W, eb3, invc_t)

    eo_t = jnp.swapaxes(eo, 0, 1)

    num_lb3 = L // LB3
    out = pl.pallas_call(
        _phase3_kernel,
        grid=(B, num_lb3),
        in_specs=[
            pl.BlockSpec((1, E, LB3), lambda b, l: (b, 0, l)),
            pl.BlockSpec((1, E, D), lambda b, l: (b, 0, 0)),
        ],
        out_specs=pl.BlockSpec((1, LB3, D), lambda b, l: (b, l, 0)),
        out_shape=jax.ShapeDtypeStruct((B, L, D), jnp.float32),
    )(disp, eo_t)

    util = jnp.sum(counts2, axis=0) / (B * L)
    diversity_loss = -jnp.var(util, ddof=1) * 0.01
    mean_gate = gates[:, :, 0] / L
    aux_loss = jnp.var(mean_gate) * E + diversity_loss
    avg_ent = jnp.sum(ents[:, 0, 0]) / (B * L)
    aux_loss = aux_loss + (avg_ent - ENTROPY_THRESHOLD) ** 2 * 0.01
    return (out, aux_loss)
